# baseline (speedup 1.0000x reference)
import functools

import jax
import jax.numpy as jnp
from jax import lax
from jax.experimental import pallas as pl
from jax.experimental.pallas import tpu as pltpu
from jax.experimental.pallas import tpu_sc as plsc

N = 10000
E = 320000
D = 128

NC = 2
NS = 16
NW = NC * NS
C = 80
NB = 3
NCW = (E // NW) // C
NP = N + 8
NT = 10
RPT = N // NT

_mesh = plsc.VectorSubcoreMesh(core_axis_name="c", subcore_axis_name="s")


@functools.partial(
    pl.kernel,
    out_type=[
        jax.ShapeDtypeStruct((NC, N, D), jnp.float32),
        jax.ShapeDtypeStruct((NC, NP), jnp.float32),
    ],
    mesh=_mesh,
    scratch_types=[
        pltpu.VMEM((NB, C), jnp.int32),
        pltpu.VMEM((NB, C), jnp.int32),
        pltpu.VMEM((NB, C, D), jnp.float32),
        pltpu.VMEM((C,), jnp.float32),
        pltpu.VMEM_SHARED((NP, D), jnp.float32),
        pltpu.VMEM_SHARED((NP,), jnp.float32),
        [pltpu.SemaphoreType.DMA] * NB,
        [pltpu.SemaphoreType.DMA] * NB,
        [pltpu.SemaphoreType.DMA] * NB,
    ],
)
def _sc_scatter(h_hbm, src_hbm, dst_hbm, zcnt_hbm,
                out_hbm, cnt_hbm,
                src_v, dst_v, rows_v, ones_v, acc_s, cnt_s,
                sem_i, sem_g, sem_s):
    cid = lax.axis_index("c")
    sid = lax.axis_index("s")
    base_w = (sid * NC + cid) * (NCW * C)

    _offs = list(range(0, C - 15, 16))
    if C % 16:
        _offs.append(C - 16)
    for off in _offs:
        ones_v[pl.ds(off, 16)] = jnp.ones((16,), jnp.float32)

    def zrow_body(r, carry):
        for j in range(D // 16):
            rows_v[0, r, pl.ds(j * 16, 16)] = jnp.zeros((16,), jnp.float32)
        return carry

    lax.fori_loop(0, C, zrow_body, 0)

    @pl.when(sid < NT)
    def _():
        def zcopy(k, carry):
            pltpu.sync_copy(
                rows_v.at[0],
                acc_s.at[pl.ds(pl.multiple_of(sid * RPT + k * C, 8), C)])
            return carry
        lax.fori_loop(0, RPT // C, zcopy, 0)
        if RPT % C:
            pltpu.sync_copy(
                rows_v.at[0, pl.ds(0, RPT % C)],
                acc_s.at[pl.ds(pl.multiple_of(
                    sid * RPT + (RPT // C) * C, 8), RPT % C)])

    @pl.when(sid == 0)
    def _():
        pltpu.sync_copy(zcnt_hbm, cnt_s)

    plsc.subcore_barrier()

    def idx_slice(hbm, i):
        return hbm.at[pl.ds(pl.multiple_of(base_w + i * C, 8), C)]

    def fire_idx(i, b):
        pltpu.async_copy(idx_slice(src_hbm, i), src_v.at[b], sem_i[b])
        pltpu.async_copy(idx_slice(dst_hbm, i), dst_v.at[b], sem_i[b])

    def wait_idx(i, b):
        pltpu.make_async_copy(idx_slice(src_hbm, i), src_v.at[b], sem_i[b]).wait()
        pltpu.make_async_copy(idx_slice(dst_hbm, i), dst_v.at[b], sem_i[b]).wait()

    def fire_gather(b):
        pltpu.async_copy(h_hbm.at[src_v.at[b]], rows_v.at[b], sem_g[b])

    def wait_gather(b):
        pltpu.make_async_copy(h_hbm.at[src_v.at[b]], rows_v.at[b], sem_g[b]).wait()

    def fire_scatter(b):
        pltpu.async_copy(ones_v, cnt_s.at[dst_v.at[b]], sem_s[b], add=True)
        pltpu.async_copy(rows_v.at[b], acc_s.at[dst_v.at[b]], sem_s[b], add=True)

    def wait_scatter(b):
        pltpu.make_async_copy(ones_v, cnt_s.at[dst_v.at[b]], sem_s[b]).wait()
        pltpu.make_async_copy(rows_v.at[b], acc_s.at[dst_v.at[b]], sem_s[b]).wait()

    for b in range(NB - 1):
        fire_idx(b, b)
    for b in range(NB - 2):
        wait_idx(b, b)
        fire_gather(b)

    def step(j, b):
        @pl.when(j >= 1)
        def _():
            wait_scatter((b - 1) % NB)

        fire_idx(j + NB - 1, (b + NB - 1) % NB)
        wait_idx(j + NB - 2, (b + NB - 2) % NB)
        fire_gather((b + NB - 2) % NB)
        wait_gather(b)
        fire_scatter(b)

    def body(g, carry):
        j0 = g * NB
        for b in range(NB):
            step(j0 + b, b)
        return carry

    full = NCW - (NB - 1)
    assert full % NB == 0
    lax.fori_loop(0, full // NB, body, 0)
    for j in range(full, NCW):
        b = j % NB
        wait_scatter((b - 1) % NB)
        if j + NB - 2 < NCW:
            wait_idx(j + NB - 2, (b + NB - 2) % NB)
            fire_gather((b + NB - 2) % NB)
        wait_gather(b)
        fire_scatter(b)
    wait_scatter((NCW - 1) % NB)

    plsc.subcore_barrier()

    @pl.when(sid < NT)
    def _():
        pltpu.sync_copy(acc_s.at[pl.ds(sid * RPT, RPT)],
                        out_hbm.at[cid, pl.ds(sid * RPT, RPT)])

    @pl.when(sid == 0)
    def _():
        pltpu.sync_copy(cnt_s, cnt_hbm.at[cid])


def _mm_body(x_ref, w_ref, b_ref, h_ref, z_ref):
    acc = jnp.dot(x_ref[...], w_ref[...],
                  preferred_element_type=jnp.float32) + b_ref[...]
    h_ref[...] = acc[:, :D]
    z_ref[...] = acc[:, D:]


_R = 1000


def _matmul2(x, w_cat, b_cat):
    grid = (N // _R,)
    return pl.pallas_call(
        _mm_body,
        grid=grid,
        in_specs=[
            pl.BlockSpec((_R, D), lambda i: (i, 0)),
            pl.BlockSpec((D, 2 * D), lambda i: (0, 0)),
            pl.BlockSpec((1, 2 * D), lambda i: (0, 0)),
        ],
        out_specs=[
            pl.BlockSpec((_R, D), lambda i: (i, 0)),
            pl.BlockSpec((_R, D), lambda i: (i, 0)),
        ],
        out_shape=[
            jax.ShapeDtypeStruct((N, D), jnp.float32),
            jax.ShapeDtypeStruct((N, D), jnp.float32),
        ],
    )(x, w_cat, b_cat)


def _combine_mm_body(p_ref, cnt_ref, z_ref, w_ref, b_ref, h2_ref, z2_ref):
    cntv = cnt_ref[...]
    tot = cntv[:, 0:1] + cntv[:, 1:2]
    rcp = 1.0 / jnp.maximum(tot, 1.0)
    h1 = jax.nn.relu((p_ref[0] + p_ref[1]) * rcp + z_ref[...])
    acc = jnp.dot(h1, w_ref[...], preferred_element_type=jnp.float32) + b_ref[...]
    h2_ref[...] = acc[:, :D]
    z2_ref[...] = acc[:, D:]


def _combine_matmul(p, cnt, z, w_cat, b_cat):
    grid = (N // _R,)
    return pl.pallas_call(
        _combine_mm_body,
        grid=grid,
        in_specs=[
            pl.BlockSpec((2, _R, D), lambda i: (0, i, 0)),
            pl.BlockSpec((_R, NC), lambda i: (i, 0)),
            pl.BlockSpec((_R, D), lambda i: (i, 0)),
            pl.BlockSpec((D, 2 * D), lambda i: (0, 0)),
            pl.BlockSpec((1, 2 * D), lambda i: (0, 0)),
        ],
        out_specs=[
            pl.BlockSpec((_R, D), lambda i: (i, 0)),
            pl.BlockSpec((_R, D), lambda i: (i, 0)),
        ],
        out_shape=[
            jax.ShapeDtypeStruct((N, D), jnp.float32),
            jax.ShapeDtypeStruct((N, D), jnp.float32),
        ],
    )(p, cnt, z, w_cat, b_cat)


def _final_body(p_ref, cnt_ref, z_ref, out_ref):
    cntv = cnt_ref[...]
    tot = cntv[:, 0:1] + cntv[:, 1:2]
    rcp = 1.0 / jnp.maximum(tot, 1.0)
    out_ref[...] = (p_ref[0] + p_ref[1]) * rcp + z_ref[...]


def _final_combine(p, cnt, z):
    grid = (N // _R,)
    return pl.pallas_call(
        _final_body,
        grid=grid,
        in_specs=[
            pl.BlockSpec((2, _R, D), lambda i: (0, i, 0)),
            pl.BlockSpec((_R, NC), lambda i: (i, 0)),
            pl.BlockSpec((_R, D), lambda i: (i, 0)),
        ],
        out_specs=pl.BlockSpec((_R, D), lambda i: (i, 0)),
        out_shape=jax.ShapeDtypeStruct((N, D), jnp.float32),
    )(p, cnt, z)


def kernel(x, edge_index, W1, lin1_W, lin1_b, W2, lin2_W, lin2_b):
    src = edge_index[0].astype(jnp.int32)
    dst = edge_index[1].astype(jnp.int32)
    zcnt = jnp.zeros((NP,), jnp.float32)

    w1c = jnp.concatenate([W1, lin1_W], axis=1)
    b1c = jnp.concatenate([jnp.zeros((D,), jnp.float32), lin1_b]).reshape(1, 2 * D)
    w2c = jnp.concatenate([W2, lin2_W], axis=1)
    b2c = jnp.concatenate([jnp.zeros((D,), jnp.float32), lin2_b]).reshape(1, 2 * D)

    h1, z1 = _matmul2(x, w1c, b1c)
    p1, cnt = _sc_scatter(h1, src, dst, zcnt)
    cnt_t = jnp.transpose(cnt[:, :N])
    h2, z2 = _combine_matmul(p1, cnt_t, z1, w2c, b2c)
    p2, _cnt2 = _sc_scatter(h2, src, dst, zcnt)
    return _final_combine(p2, cnt_t, z2)


# speedup vs baseline: 1.0716x; 1.0716x over previous
import functools

import jax
import jax.numpy as jnp
from jax import lax
from jax.experimental import pallas as pl
from jax.experimental.pallas import tpu as pltpu
from jax.experimental.pallas import tpu_sc as plsc

N = 10000
E = 320000
D = 128

NC = 2
NS = 16
NW = NC * NS
C = 40
NB = 5
N0 = 250
N1 = 250
SC0E = NS * N0 * C
NP = N + 8
NT = 10
RPT = N // NT

_mesh = plsc.VectorSubcoreMesh(core_axis_name="c", subcore_axis_name="s")


@functools.partial(
    pl.kernel,
    out_type=[
        jax.ShapeDtypeStruct((NC, N, D), jnp.float32),
        jax.ShapeDtypeStruct((NC, NP), jnp.float32),
    ],
    mesh=_mesh,
    scratch_types=[
        pltpu.VMEM((NB, C), jnp.int32),
        pltpu.VMEM((NB, C), jnp.int32),
        pltpu.VMEM((NB, C, D), jnp.float32),
        pltpu.VMEM((C,), jnp.float32),
        pltpu.VMEM_SHARED((NP, D), jnp.float32),
        pltpu.VMEM_SHARED((NP,), jnp.float32),
        [pltpu.SemaphoreType.DMA] * NB,
        [pltpu.SemaphoreType.DMA] * NB,
        [pltpu.SemaphoreType.DMA] * NB,
    ],
)
def _sc_scatter(h_hbm, src_hbm, dst_hbm, zcnt_hbm,
                out_hbm, cnt_hbm,
                src_v, dst_v, rows_v, ones_v, acc_s, cnt_s,
                sem_i, sem_g, sem_s):
    cid = lax.axis_index("c")
    sid = lax.axis_index("s")
    base_w = lax.select(cid == 0, sid * (N0 * C), SC0E + sid * (N1 * C))
    ncw = lax.select(cid == 0, N0, N1)

    _offs = list(range(0, C - 15, 16))
    if C % 16:
        _offs.append(C - 16)
    for off in _offs:
        ones_v[pl.ds(off, 16)] = jnp.ones((16,), jnp.float32)

    def zrow_body(r, carry):
        for j in range(D // 16):
            rows_v[0, r, pl.ds(j * 16, 16)] = jnp.zeros((16,), jnp.float32)
        return carry

    lax.fori_loop(0, C, zrow_body, 0)

    @pl.when(sid < NT)
    def _():
        def zcopy(k, carry):
            pltpu.sync_copy(
                rows_v.at[0],
                acc_s.at[pl.ds(pl.multiple_of(sid * RPT + k * C, 8), C)])
            return carry
        lax.fori_loop(0, RPT // C, zcopy, 0)

    @pl.when(sid == 0)
    def _():
        pltpu.sync_copy(zcnt_hbm, cnt_s)

    plsc.subcore_barrier()

    def idx_slice(hbm, i):
        return hbm.at[pl.ds(pl.multiple_of(base_w + i * C, 8), C)]

    def fire_idx(i, b):
        pltpu.async_copy(idx_slice(src_hbm, i), src_v.at[b], sem_i[b])
        pltpu.async_copy(idx_slice(dst_hbm, i), dst_v.at[b], sem_i[b])

    def wait_idx(i, b):
        pltpu.make_async_copy(idx_slice(src_hbm, i), src_v.at[b], sem_i[b]).wait()
        pltpu.make_async_copy(idx_slice(dst_hbm, i), dst_v.at[b], sem_i[b]).wait()

    def fire_gather(b):
        pltpu.async_copy(h_hbm.at[src_v.at[b]], rows_v.at[b], sem_g[b])

    def wait_gather(b):
        pltpu.make_async_copy(h_hbm.at[src_v.at[b]], rows_v.at[b], sem_g[b]).wait()

    def fire_scatter(b):
        pltpu.async_copy(ones_v, cnt_s.at[dst_v.at[b]], sem_s[b], add=True)
        pltpu.async_copy(rows_v.at[b], acc_s.at[dst_v.at[b]], sem_s[b], add=True)

    def wait_scatter(b):
        pltpu.make_async_copy(ones_v, cnt_s.at[dst_v.at[b]], sem_s[b]).wait()
        pltpu.make_async_copy(rows_v.at[b], acc_s.at[dst_v.at[b]], sem_s[b]).wait()

    for b in range(NB - 1):
        fire_idx(b, b)
    for b in range(NB - 2):
        wait_idx(b, b)
        fire_gather(b)

    def step(j, b):
        @pl.when(j >= 1)
        def _():
            wait_scatter((b - 1) % NB)

        @pl.when(j + NB - 1 < ncw)
        def _():
            fire_idx(j + NB - 1, (b + NB - 1) % NB)

        @pl.when(j + NB - 2 < ncw)
        def _():
            wait_idx(j + NB - 2, (b + NB - 2) % NB)
            fire_gather((b + NB - 2) % NB)

        wait_gather(b)
        fire_scatter(b)

    def body(g, carry):
        j0 = g * NB
        for b in range(NB):
            step(j0 + b, b)
        return carry

    lax.fori_loop(0, lax.select(cid == 0, N0 // NB, N1 // NB), body, 0)
    wait_scatter((NB - 1) % NB)

    plsc.subcore_barrier()

    @pl.when(sid < NT)
    def _():
        pltpu.sync_copy(acc_s.at[pl.ds(sid * RPT, RPT)],
                        out_hbm.at[cid, pl.ds(sid * RPT, RPT)])

    @pl.when(sid == 0)
    def _():
        pltpu.sync_copy(cnt_s, cnt_hbm.at[cid])


_R = 1000


def _aggr(p_ref, cnt_ref):
    cntv = cnt_ref[...]
    tot = cntv[:, 0:1] + cntv[:, 1:2]
    rcp = 1.0 / jnp.maximum(tot, 1.0)
    return (p_ref[0] + p_ref[1]) * rcp


def _linear_body(x_ref, w_ref, b_ref, z_ref):
    z_ref[...] = jnp.dot(x_ref[...], w_ref[...],
                         preferred_element_type=jnp.float32) + b_ref[...]


def _linear(x, w, b):
    return pl.pallas_call(
        _linear_body,
        grid=(N // _R,),
        in_specs=[
            pl.BlockSpec((_R, D), lambda i: (i, 0)),
            pl.BlockSpec((D, D), lambda i: (0, 0)),
            pl.BlockSpec((1, D), lambda i: (0, 0)),
        ],
        out_specs=pl.BlockSpec((_R, D), lambda i: (i, 0)),
        out_shape=jax.ShapeDtypeStruct((N, D), jnp.float32),
    )(x, w, b)


def _mid_body(p_ref, cnt_ref, z_ref, w1_ref, w2_ref, b2_ref, h1_ref, z2_ref):
    h1 = jax.nn.relu(
        jnp.dot(_aggr(p_ref, cnt_ref), w1_ref[...],
                preferred_element_type=jnp.float32) + z_ref[...])
    h1_ref[...] = h1
    z2_ref[...] = jnp.dot(h1, w2_ref[...],
                          preferred_element_type=jnp.float32) + b2_ref[...]


def _mid(p, cnt_t, z1, w1, w2, b2):
    return pl.pallas_call(
        _mid_body,
        grid=(N // _R,),
        in_specs=[
            pl.BlockSpec((2, _R, D), lambda i: (0, i, 0)),
            pl.BlockSpec((_R, NC), lambda i: (i, 0)),
            pl.BlockSpec((_R, D), lambda i: (i, 0)),
            pl.BlockSpec((D, D), lambda i: (0, 0)),
            pl.BlockSpec((D, D), lambda i: (0, 0)),
            pl.BlockSpec((1, D), lambda i: (0, 0)),
        ],
        out_specs=[
            pl.BlockSpec((_R, D), lambda i: (i, 0)),
            pl.BlockSpec((_R, D), lambda i: (i, 0)),
        ],
        out_shape=[
            jax.ShapeDtypeStruct((N, D), jnp.float32),
            jax.ShapeDtypeStruct((N, D), jnp.float32),
        ],
    )(p, cnt_t, z1, w1, w2, b2)


def _fin_body(p_ref, cnt_ref, z_ref, w_ref, out_ref):
    out_ref[...] = jnp.dot(_aggr(p_ref, cnt_ref), w_ref[...],
                           preferred_element_type=jnp.float32) + z_ref[...]


def _fin(p, cnt_t, z2, w2):
    return pl.pallas_call(
        _fin_body,
        grid=(N // _R,),
        in_specs=[
            pl.BlockSpec((2, _R, D), lambda i: (0, i, 0)),
            pl.BlockSpec((_R, NC), lambda i: (i, 0)),
            pl.BlockSpec((_R, D), lambda i: (i, 0)),
            pl.BlockSpec((D, D), lambda i: (0, 0)),
        ],
        out_specs=pl.BlockSpec((_R, D), lambda i: (i, 0)),
        out_shape=jax.ShapeDtypeStruct((N, D), jnp.float32),
    )(p, cnt_t, z2, w2)


def kernel(x, edge_index, W1, lin1_W, lin1_b, W2, lin2_W, lin2_b):
    src = edge_index[0].astype(jnp.int32)
    dst = edge_index[1].astype(jnp.int32)
    zcnt = jnp.zeros((NP,), jnp.float32)
    b1 = lin1_b.reshape(1, D)
    b2 = lin2_b.reshape(1, D)

    p1, cnt = _sc_scatter(x, src, dst, zcnt)
    z1 = _linear(x, lin1_W, b1)
    cnt_t = jnp.transpose(cnt[:, :N])
    h1, z2 = _mid(p1, cnt_t, z1, W1, lin2_W, b2)
    p2, _cnt2 = _sc_scatter(h1, src, dst, zcnt)
    return _fin(p2, cnt_t, z2, W2)


# speedup vs baseline: 1.0776x; 1.0056x over previous
import functools

import jax
import jax.numpy as jnp
from jax import lax
from jax.experimental import pallas as pl
from jax.experimental.pallas import tpu as pltpu
from jax.experimental.pallas import tpu_sc as plsc

N = 10000
E = 320000
D = 128

NC = 2
NS = 16
NW = NC * NS
C = 40
NB = 5
N0 = 250
N1 = 250
SC0E = NS * N0 * C
NP = N + 8
NT = 10
RPT = N // NT

_mesh = plsc.VectorSubcoreMesh(core_axis_name="c", subcore_axis_name="s")


@functools.partial(
    pl.kernel,
    out_type=[
        jax.ShapeDtypeStruct((NC, N, D), jnp.float32),
        jax.ShapeDtypeStruct((NC, NP), jnp.float32),
    ],
    mesh=_mesh,
    scratch_types=[
        pltpu.VMEM((NB, C), jnp.int32),
        pltpu.VMEM((NB, C), jnp.int32),
        pltpu.VMEM((NB, C, D), jnp.float32),
        pltpu.VMEM((C,), jnp.float32),
        pltpu.VMEM_SHARED((NP, D), jnp.float32),
        pltpu.VMEM_SHARED((NP,), jnp.float32),
        [pltpu.SemaphoreType.DMA] * NB,
        [pltpu.SemaphoreType.DMA] * NB,
        [pltpu.SemaphoreType.DMA] * NB,
    ],
)
def _sc_scatter(h_hbm, src_hbm, dst_hbm, zcnt_hbm,
                out_hbm, cnt_hbm,
                src_v, dst_v, rows_v, ones_v, acc_s, cnt_s,
                sem_i, sem_g, sem_s):
    cid = lax.axis_index("c")
    sid = lax.axis_index("s")
    base_w = lax.select(cid == 0, sid * (N0 * C), SC0E + sid * (N1 * C))
    ncw = lax.select(cid == 0, N0, N1)

    _offs = list(range(0, C - 15, 16))
    if C % 16:
        _offs.append(C - 16)
    for off in _offs:
        ones_v[pl.ds(off, 16)] = jnp.ones((16,), jnp.float32)

    def zrow_body(r, carry):
        for j in range(D // 16):
            rows_v[0, r, pl.ds(j * 16, 16)] = jnp.zeros((16,), jnp.float32)
        return carry

    lax.fori_loop(0, C, zrow_body, 0)

    @pl.when(sid < NT)
    def _():
        def zcopy(k, carry):
            pltpu.sync_copy(
                rows_v.at[0],
                acc_s.at[pl.ds(pl.multiple_of(sid * RPT + k * C, 8), C)])
            return carry
        lax.fori_loop(0, RPT // C, zcopy, 0)

    @pl.when(sid == 0)
    def _():
        pltpu.sync_copy(zcnt_hbm, cnt_s)

    plsc.subcore_barrier()

    def idx_slice(hbm, i):
        return hbm.at[pl.ds(pl.multiple_of(base_w + i * C, 8), C)]

    def fire_idx(i, b):
        pltpu.async_copy(idx_slice(src_hbm, i), src_v.at[b], sem_i[b])
        pltpu.async_copy(idx_slice(dst_hbm, i), dst_v.at[b], sem_i[b])

    def wait_idx(i, b):
        pltpu.make_async_copy(idx_slice(src_hbm, i), src_v.at[b], sem_i[b]).wait()
        pltpu.make_async_copy(idx_slice(dst_hbm, i), dst_v.at[b], sem_i[b]).wait()

    def fire_gather(b):
        pltpu.async_copy(h_hbm.at[src_v.at[b]], rows_v.at[b], sem_g[b])

    def wait_gather(b):
        pltpu.make_async_copy(h_hbm.at[src_v.at[b]], rows_v.at[b], sem_g[b]).wait()

    def fire_scatter(b):
        pltpu.async_copy(ones_v, cnt_s.at[dst_v.at[b]], sem_s[b], add=True)
        pltpu.async_copy(rows_v.at[b], acc_s.at[dst_v.at[b]], sem_s[b], add=True)

    def wait_scatter(b):
        pltpu.make_async_copy(ones_v, cnt_s.at[dst_v.at[b]], sem_s[b]).wait()
        pltpu.make_async_copy(rows_v.at[b], acc_s.at[dst_v.at[b]], sem_s[b]).wait()

    for b in range(NB - 1):
        fire_idx(b, b)
    for b in range(NB - 2):
        wait_idx(b, b)
        fire_gather(b)

    def step(j, b):
        @pl.when(j >= 1)
        def _():
            wait_scatter((b - 1) % NB)

        @pl.when(j + NB - 1 < ncw)
        def _():
            fire_idx(j + NB - 1, (b + NB - 1) % NB)

        @pl.when(j + NB - 2 < ncw)
        def _():
            wait_idx(j + NB - 2, (b + NB - 2) % NB)
            fire_gather((b + NB - 2) % NB)

        wait_gather(b)
        fire_scatter(b)

    def body(g, carry):
        j0 = g * NB
        for b in range(NB):
            step(j0 + b, b)
        return carry

    lax.fori_loop(0, lax.select(cid == 0, N0 // NB, N1 // NB), body, 0)
    wait_scatter((NB - 1) % NB)

    plsc.subcore_barrier()

    @pl.when(sid < NT)
    def _():
        pltpu.sync_copy(acc_s.at[pl.ds(sid * RPT, RPT)],
                        out_hbm.at[cid, pl.ds(sid * RPT, RPT)])

    @pl.when(sid == 0)
    def _():
        pltpu.sync_copy(cnt_s, cnt_hbm.at[cid])


_R = 1000


def _aggr(p_ref, cnt_ref):
    cntv = cnt_ref[...]
    tot = cntv[:, 0:1] + cntv[:, 1:2]
    rcp = 1.0 / jnp.maximum(tot, 1.0)
    return (p_ref[0] + p_ref[1]) * rcp


def _linear_body(x_ref, w_ref, b_ref, z_ref):
    z_ref[...] = jnp.dot(x_ref[...], w_ref[...],
                         preferred_element_type=jnp.float32) + b_ref[...]


def _linear(x, w, b):
    return pl.pallas_call(
        _linear_body,
        grid=(N // _R,),
        in_specs=[
            pl.BlockSpec((_R, D), lambda i: (i, 0)),
            pl.BlockSpec((D, D), lambda i: (0, 0)),
            pl.BlockSpec((1, D), lambda i: (0, 0)),
        ],
        out_specs=pl.BlockSpec((_R, D), lambda i: (i, 0)),
        out_shape=jax.ShapeDtypeStruct((N, D), jnp.float32),
    )(x, w, b)


def _mid_body(p_ref, cnt_ref, z_ref, w1_ref, h1_ref):
    h1_ref[...] = jax.nn.relu(
        jnp.dot(_aggr(p_ref, cnt_ref), w1_ref[...],
                preferred_element_type=jnp.float32) + z_ref[...])


def _mid(p, cnt_t, z1, w1):
    return pl.pallas_call(
        _mid_body,
        grid=(N // _R,),
        in_specs=[
            pl.BlockSpec((2, _R, D), lambda i: (0, i, 0)),
            pl.BlockSpec((_R, NC), lambda i: (i, 0)),
            pl.BlockSpec((_R, D), lambda i: (i, 0)),
            pl.BlockSpec((D, D), lambda i: (0, 0)),
        ],
        out_specs=pl.BlockSpec((_R, D), lambda i: (i, 0)),
        out_shape=jax.ShapeDtypeStruct((N, D), jnp.float32),
    )(p, cnt_t, z1, w1)


def _fin_body(p_ref, cnt_ref, z_ref, w_ref, out_ref):
    out_ref[...] = jnp.dot(_aggr(p_ref, cnt_ref), w_ref[...],
                           preferred_element_type=jnp.float32) + z_ref[...]


def _fin(p, cnt_t, z2, w2):
    return pl.pallas_call(
        _fin_body,
        grid=(N // _R,),
        in_specs=[
            pl.BlockSpec((2, _R, D), lambda i: (0, i, 0)),
            pl.BlockSpec((_R, NC), lambda i: (i, 0)),
            pl.BlockSpec((_R, D), lambda i: (i, 0)),
            pl.BlockSpec((D, D), lambda i: (0, 0)),
        ],
        out_specs=pl.BlockSpec((_R, D), lambda i: (i, 0)),
        out_shape=jax.ShapeDtypeStruct((N, D), jnp.float32),
    )(p, cnt_t, z2, w2)


def kernel(x, edge_index, W1, lin1_W, lin1_b, W2, lin2_W, lin2_b):
    src = edge_index[0].astype(jnp.int32)
    dst = edge_index[1].astype(jnp.int32)
    zcnt = jnp.zeros((NP,), jnp.float32)
    b1 = lin1_b.reshape(1, D)
    b2 = lin2_b.reshape(1, D)

    p1, cnt = _sc_scatter(x, src, dst, zcnt)
    z1 = _linear(x, lin1_W, b1)
    cnt_t = jnp.transpose(cnt[:, :N])
    h1 = _mid(p1, cnt_t, z1, W1)
    p2, _cnt2 = _sc_scatter(h1, src, dst, zcnt)
    z2 = _linear(h1, lin2_W, b2)
    return _fin(p2, cnt_t, z2, W2)


# speedup vs baseline: 1.1167x; 1.0363x over previous
import functools

import jax
import jax.numpy as jnp
from jax import lax
from jax.experimental import pallas as pl
from jax.experimental.pallas import tpu as pltpu
from jax.experimental.pallas import tpu_sc as plsc

N = 10000
E = 320000
D = 128

NC = 2
NS = 16
NW = NC * NS
C = 40
NB = 5
N0 = 250
N1 = 250
SC0E = NS * N0 * C
NP = N + 8
NT = 10
RPT = N // NT

_mesh = plsc.VectorSubcoreMesh(core_axis_name="c", subcore_axis_name="s")


@functools.partial(
    pl.kernel,
    out_type=[
        jax.ShapeDtypeStruct((NC, N, D), jnp.float32),
        jax.ShapeDtypeStruct((NC, NP), jnp.float32),
    ],
    mesh=_mesh,
    scratch_types=[
        pltpu.VMEM((NB, C), jnp.int32),
        pltpu.VMEM((NB, C), jnp.int32),
        pltpu.VMEM((NB, C, D), jnp.float32),
        pltpu.VMEM((C,), jnp.float32),
        pltpu.VMEM_SHARED((NP, D), jnp.float32),
        pltpu.VMEM_SHARED((NP,), jnp.float32),
        [pltpu.SemaphoreType.DMA] * NB,
        [pltpu.SemaphoreType.DMA] * NB,
        [pltpu.SemaphoreType.DMA] * NB,
    ],
)
def _sc_scatter(h_hbm, ei_hbm, zcnt_hbm,
                out_hbm, cnt_hbm,
                src_v, dst_v, rows_v, ones_v, acc_s, cnt_s,
                sem_i, sem_g, sem_s):
    cid = lax.axis_index("c")
    sid = lax.axis_index("s")
    base_w = lax.select(cid == 0, sid * (N0 * C), SC0E + sid * (N1 * C))
    ncw = lax.select(cid == 0, N0, N1)

    _offs = list(range(0, C - 15, 16))
    if C % 16:
        _offs.append(C - 16)
    for off in _offs:
        ones_v[pl.ds(off, 16)] = jnp.ones((16,), jnp.float32)

    def zrow_body(r, carry):
        for j in range(D // 16):
            rows_v[0, r, pl.ds(j * 16, 16)] = jnp.zeros((16,), jnp.float32)
        return carry

    lax.fori_loop(0, C, zrow_body, 0)

    @pl.when(sid < NT)
    def _():
        def zcopy(k, carry):
            pltpu.sync_copy(
                rows_v.at[0],
                acc_s.at[pl.ds(pl.multiple_of(sid * RPT + k * C, 8), C)])
            return carry
        lax.fori_loop(0, RPT // C, zcopy, 0)

    @pl.when(sid == 0)
    def _():
        pltpu.sync_copy(zcnt_hbm, cnt_s)

    plsc.subcore_barrier()

    def idx_slice(off, i):
        return ei_hbm.at[pl.ds(pl.multiple_of(off + base_w + i * C, 8), C)]

    def fire_idx(i, b):
        pltpu.async_copy(idx_slice(0, i), src_v.at[b], sem_i[b])
        pltpu.async_copy(idx_slice(E, i), dst_v.at[b], sem_i[b])

    def wait_idx(i, b):
        pltpu.make_async_copy(idx_slice(0, i), src_v.at[b], sem_i[b]).wait()
        pltpu.make_async_copy(idx_slice(E, i), dst_v.at[b], sem_i[b]).wait()

    def fire_gather(b):
        pltpu.async_copy(h_hbm.at[src_v.at[b]], rows_v.at[b], sem_g[b])

    def wait_gather(b):
        pltpu.make_async_copy(h_hbm.at[src_v.at[b]], rows_v.at[b], sem_g[b]).wait()

    def fire_scatter(b):
        pltpu.async_copy(ones_v, cnt_s.at[dst_v.at[b]], sem_s[b], add=True)
        pltpu.async_copy(rows_v.at[b], acc_s.at[dst_v.at[b]], sem_s[b], add=True)

    def wait_scatter(b):
        pltpu.make_async_copy(ones_v, cnt_s.at[dst_v.at[b]], sem_s[b]).wait()
        pltpu.make_async_copy(rows_v.at[b], acc_s.at[dst_v.at[b]], sem_s[b]).wait()

    for b in range(NB - 1):
        fire_idx(b, b)
    for b in range(NB - 2):
        wait_idx(b, b)
        fire_gather(b)

    def step(j, b):
        @pl.when(j >= 1)
        def _():
            wait_scatter((b - 1) % NB)

        @pl.when(j + NB - 1 < ncw)
        def _():
            fire_idx(j + NB - 1, (b + NB - 1) % NB)

        @pl.when(j + NB - 2 < ncw)
        def _():
            wait_idx(j + NB - 2, (b + NB - 2) % NB)
            fire_gather((b + NB - 2) % NB)

        wait_gather(b)
        fire_scatter(b)

    def body(g, carry):
        j0 = g * NB
        for b in range(NB):
            step(j0 + b, b)
        return carry

    lax.fori_loop(0, lax.select(cid == 0, N0 // NB, N1 // NB), body, 0)
    wait_scatter((NB - 1) % NB)

    plsc.subcore_barrier()

    @pl.when(sid < NT)
    def _():
        pltpu.sync_copy(acc_s.at[pl.ds(sid * RPT, RPT)],
                        out_hbm.at[cid, pl.ds(sid * RPT, RPT)])

    @pl.when(sid == 0)
    def _():
        pltpu.sync_copy(cnt_s, cnt_hbm.at[cid])


_R = 1000


def _aggr(p_ref, cnt_ref):
    cntv = cnt_ref[...]
    tot = cntv[:, 0:1] + cntv[:, 1:2]
    rcp = 1.0 / jnp.maximum(tot, 1.0)
    return (p_ref[0] + p_ref[1]) * rcp


def _linear_body(x_ref, w_ref, b_ref, z_ref):
    z_ref[...] = jnp.dot(x_ref[...], w_ref[...],
                         preferred_element_type=jnp.float32) + b_ref[...]


def _linear(x, w, b):
    return pl.pallas_call(
        _linear_body,
        grid=(N // _R,),
        in_specs=[
            pl.BlockSpec((_R, D), lambda i: (i, 0)),
            pl.BlockSpec((D, D), lambda i: (0, 0)),
            pl.BlockSpec((1, D), lambda i: (0, 0)),
        ],
        out_specs=pl.BlockSpec((_R, D), lambda i: (i, 0)),
        out_shape=jax.ShapeDtypeStruct((N, D), jnp.float32),
    )(x, w, b)


def _mid_body(p_ref, cnt_ref, z_ref, w1_ref, h1_ref):
    h1_ref[...] = jax.nn.relu(
        jnp.dot(_aggr(p_ref, cnt_ref), w1_ref[...],
                preferred_element_type=jnp.float32) + z_ref[...])


def _mid(p, cnt_t, z1, w1):
    return pl.pallas_call(
        _mid_body,
        grid=(N // _R,),
        in_specs=[
            pl.BlockSpec((2, _R, D), lambda i: (0, i, 0)),
            pl.BlockSpec((_R, NC), lambda i: (i, 0)),
            pl.BlockSpec((_R, D), lambda i: (i, 0)),
            pl.BlockSpec((D, D), lambda i: (0, 0)),
        ],
        out_specs=pl.BlockSpec((_R, D), lambda i: (i, 0)),
        out_shape=jax.ShapeDtypeStruct((N, D), jnp.float32),
    )(p, cnt_t, z1, w1)


def _fin_body(p_ref, cnt_ref, z_ref, w_ref, out_ref):
    out_ref[...] = jnp.dot(_aggr(p_ref, cnt_ref), w_ref[...],
                           preferred_element_type=jnp.float32) + z_ref[...]


def _fin(p, cnt_t, z2, w2):
    return pl.pallas_call(
        _fin_body,
        grid=(N // _R,),
        in_specs=[
            pl.BlockSpec((2, _R, D), lambda i: (0, i, 0)),
            pl.BlockSpec((_R, NC), lambda i: (i, 0)),
            pl.BlockSpec((_R, D), lambda i: (i, 0)),
            pl.BlockSpec((D, D), lambda i: (0, 0)),
        ],
        out_specs=pl.BlockSpec((_R, D), lambda i: (i, 0)),
        out_shape=jax.ShapeDtypeStruct((N, D), jnp.float32),
    )(p, cnt_t, z2, w2)


def kernel(x, edge_index, W1, lin1_W, lin1_b, W2, lin2_W, lin2_b):
    ei = jnp.reshape(edge_index.astype(jnp.int32), (2 * E,))
    zcnt = jnp.zeros((NP,), jnp.float32)
    b1 = lin1_b.reshape(1, D)
    b2 = lin2_b.reshape(1, D)

    p1, cnt = _sc_scatter(x, ei, zcnt)
    z1 = _linear(x, lin1_W, b1)
    cnt_t = jnp.transpose(cnt[:, :N])
    h1 = _mid(p1, cnt_t, z1, W1)
    p2, _cnt2 = _sc_scatter(h1, ei, zcnt)
    z2 = _linear(h1, lin2_W, b2)
    return _fin(p2, cnt_t, z2, W2)
